# Initial kernel scaffold; baseline (speedup 1.0000x reference)
import jax
import jax.numpy as jnp
from jax.experimental import pallas as pl


def kernel(reconstructed_image, target_image):
    raise NotImplementedError("write your pallas kernel here")


# speedup vs baseline: 48.3395x; 48.3395x over previous
import jax
import jax.numpy as jnp
from jax import lax
from jax.experimental import pallas as pl
from jax.experimental.pallas import tpu as pltpu
from jax.experimental.pallas import tpu_sc as plsc

_TIMESTEPS = 1000
_B, _H, _W = 8, 512, 512
_ROWS = _B * _H
_L = 16
_NC, _NS = 2, 16
_NW = _NC * _NS
_BLK = _L
_NBLK = _ROWS // (_NW * _BLK)
_NBINS = 1024


def _sc_body(rec_hbm, tgt_hbm, out_hbm, rec_v, tgt_v, histr_v, histt_v, acc_v):
    wid = lax.axis_index("s") * _NC + lax.axis_index("c")
    lanes = lax.iota(jnp.int32, _L)
    zero16 = jnp.zeros((_L,), jnp.float32)
    lane_hist_base = lanes * _NBINS
    lane_row_base = lanes * _W

    def block_body(blk, acc):
        r0 = (wid * _NBLK + blk) * _BLK
        pltpu.sync_copy(rec_hbm.at[pl.ds(r0 * _W, _BLK * _W)], rec_v)
        pltpu.sync_copy(tgt_hbm.at[pl.ds(r0 * _W, _BLK * _W)], tgt_v)

        def init_body(i, carry):
            histr_v[pl.ds(i * _L, _L)] = zero16
            histt_v[pl.ds(i * _L, _L)] = zero16
            return carry

        lax.fori_loop(0, _BLK * _NBINS // _L, init_body, 0)

        def scat_body(step, carry):
            y = step - lanes
            valid = (y >= 0) & (y < _W)
            yc = jnp.clip(y, 0, _W - 1)
            addr = lane_row_base + yc
            rv = plsc.load_gather(rec_v, [addr])
            tv = plsc.load_gather(tgt_v, [addr])
            yf = y.astype(jnp.float32)
            qr = (rv * 1000.0).astype(jnp.int32) - 1
            qr = jnp.where(qr < 0, qr + _TIMESTEPS, qr)
            plsc.store_scatter(histr_v, [lane_hist_base + qr], yf,
                               mask=valid & (rv != 0.0))
            qt = (tv * 1000.0).astype(jnp.int32) - 1
            qt = jnp.where(qt < 0, qt + _TIMESTEPS, qt)
            plsc.store_scatter(histt_v, [lane_hist_base + qt], yf,
                               mask=valid & (tv != 0.0))
            return carry

        lax.fori_loop(0, _W + _L, scat_body, 0)

        def diff_body(i, a):
            d = histr_v[pl.ds(i * _L, _L)] - histt_v[pl.ds(i * _L, _L)]
            return a + d * d

        return lax.fori_loop(0, _BLK * _NBINS // _L, diff_body, acc)

    acc = lax.fori_loop(0, _NBLK, block_body, jnp.zeros((_L,), jnp.float32))
    acc_v[...] = acc
    pltpu.sync_copy(acc_v, out_hbm.at[pl.ds(wid * _L, _L)])


def kernel(reconstructed_image, target_image):
    rec = reconstructed_image.reshape(_ROWS * _W)
    tgt = target_image.reshape(_ROWS * _W)
    mesh = plsc.VectorSubcoreMesh(core_axis_name="c", subcore_axis_name="s")
    partials = pl.kernel(
        _sc_body,
        out_type=jax.ShapeDtypeStruct((_NW * _L,), jnp.float32),
        mesh=mesh,
        scratch_types=[
            pltpu.VMEM((_BLK * _W,), jnp.float32),
            pltpu.VMEM((_BLK * _W,), jnp.float32),
            pltpu.VMEM((_BLK * _NBINS,), jnp.float32),
            pltpu.VMEM((_BLK * _NBINS,), jnp.float32),
            pltpu.VMEM((_L,), jnp.float32),
        ],
        compiler_params=pltpu.CompilerParams(needs_layout_passes=False),
    )(rec, tgt)
    return jnp.sum(partials) / jnp.float32(_B * _H * _TIMESTEPS)


# speedup vs baseline: 149.3596x; 3.0898x over previous
import jax
import jax.numpy as jnp
from jax import lax
from jax.experimental import pallas as pl
from jax.experimental.pallas import tpu as pltpu
from jax.experimental.pallas import tpu_sc as plsc

_B, _H, _W = 8, 512, 512
_ROWS = _B * _H
_L = 16
_NC, _NS = 2, 16
_NW = _NC * _NS
_BLK = _L
_NBLK = _ROWS // (_NW * _BLK)
_NBINS = 1024


def _sc_body(rec_hbm, tgt_hbm, out_hbm,
             rec_a, tgt_a, rec_b, tgt_b, histr_v, histt_v, acc_v,
             sem_a, sem_b):
    wid = lax.axis_index("s") * _NC + lax.axis_index("c")
    lanes = lax.iota(jnp.int32, _L)
    zero16 = jnp.zeros((_L,), jnp.float32)
    lane_hist = lanes * _NBINS

    bufs = ((rec_a, tgt_a, sem_a), (rec_b, tgt_b, sem_b))

    def start_block_dma(blk):
        rv, tv, sem = bufs[blk % 2]
        off = wid * (_NBLK * _BLK) + blk * _BLK
        hr = pltpu.async_copy(rec_hbm.at[pl.ds(off, _BLK)], rv, sem)
        ht = pltpu.async_copy(tgt_hbm.at[pl.ds(off, _BLK)], tv, sem)
        return hr, ht

    def init_body(i, carry):
        histr_v[pl.ds(i * _L, _L)] = zero16
        histt_v[pl.ds(i * _L, _L)] = zero16
        return carry

    pending = start_block_dma(0)
    lax.fori_loop(0, _BLK * _NBINS // _L, init_body, 0)

    acc0 = jnp.zeros((_L,), jnp.float32)
    acc1 = jnp.zeros((_L,), jnp.float32)

    for blk in range(_NBLK):
        rec_v, tgt_v, _ = bufs[blk % 2]
        pending[0].wait()
        pending[1].wait()
        if blk + 1 < _NBLK:
            pending = start_block_dma(blk + 1)

        def scat_body(i, carry, rec_v=rec_v, tgt_v=tgt_v, edge=True, u=8):
            ys, rvs, tvs = [], [], []
            for k in range(u):
                y = (i * u + k) - lanes
                yc = jnp.clip(y, 0, _W - 1) if edge else y
                rvs.append(plsc.load_gather(rec_v, [lanes, yc]))
                tvs.append(plsc.load_gather(tgt_v, [lanes, yc]))
                ys.append(y)
            for k in range(u):
                y, rv, tv = ys[k], rvs[k], tvs[k]
                yf = y.astype(jnp.float32)
                mr, mt = rv > 0.0, tv > 0.0
                if edge:
                    valid = (y >= 0) & (y < _W)
                    mr, mt = valid & mr, valid & mt
                tr = (rv * 1000.0).astype(jnp.int32)
                plsc.store_scatter(histr_v, [lane_hist + tr], yf, mask=mr)
                tt = (tv * 1000.0).astype(jnp.int32)
                plsc.store_scatter(histt_v, [lane_hist + tt], yf, mask=mt)
            return carry

        def scat_mid(i, carry, rec_v=rec_v, tgt_v=tgt_v):
            return scat_body(i, carry, rec_v=rec_v, tgt_v=tgt_v, edge=False)

        lax.fori_loop(0, 2, scat_body, 0)
        lax.fori_loop(2, 64, scat_mid, 0)
        lax.fori_loop(64, 66, scat_body, 0)

        def diff_body(i, carry):
            a0, a1 = carry
            for j in range(4):
                sl = pl.ds(i * (4 * _L) + j * _L, _L)
                d = histr_v[sl] - histt_v[sl]
                histr_v[sl] = zero16
                histt_v[sl] = zero16
                if j % 2 == 0:
                    a0 = a0 + d * d
                else:
                    a1 = a1 + d * d
            return (a0, a1)

        acc0, acc1 = lax.fori_loop(0, _BLK * _NBINS // (4 * _L), diff_body,
                                   (acc0, acc1))

    acc_v[...] = acc0 + acc1
    pltpu.sync_copy(acc_v, out_hbm.at[pl.ds(wid * _L, _L)])


def kernel(reconstructed_image, target_image):
    rec = reconstructed_image.reshape(_ROWS, _W)
    tgt = target_image.reshape(_ROWS, _W)
    mesh = plsc.VectorSubcoreMesh(core_axis_name="c", subcore_axis_name="s")
    partials = pl.kernel(
        _sc_body,
        out_type=jax.ShapeDtypeStruct((_NW * _L,), jnp.float32),
        mesh=mesh,
        scratch_types=[
            pltpu.VMEM((_BLK, _W), jnp.float32),
            pltpu.VMEM((_BLK, _W), jnp.float32),
            pltpu.VMEM((_BLK, _W), jnp.float32),
            pltpu.VMEM((_BLK, _W), jnp.float32),
            pltpu.VMEM((_BLK * _NBINS,), jnp.float32),
            pltpu.VMEM((_BLK * _NBINS,), jnp.float32),
            pltpu.VMEM((_L,), jnp.float32),
            pltpu.SemaphoreType.DMA,
            pltpu.SemaphoreType.DMA,
        ],
        compiler_params=pltpu.CompilerParams(needs_layout_passes=False,
                                             use_tc_tiling_on_sc=True),
    )(rec, tgt)
    return jnp.sum(partials) / jnp.float32(_B * _H * 1000)
